# Initial kernel scaffold; baseline (speedup 1.0000x reference)
import jax
import jax.numpy as jnp
from jax.experimental import pallas as pl


def kernel(x, pos, edge_index, W_lin_in, W_lin, W_src, W_dst, pos_W1, pos_b1, pos_W2, pos_b2, attn_W1, attn_b1, attn_W2, attn_b2, W_lin_out):
    raise NotImplementedError("write your pallas kernel here")


# speedup vs baseline: 1.3654x; 1.3654x over previous
import functools

import jax
import jax.numpy as jnp
from jax import lax
from jax.experimental import pallas as pl
from jax.experimental.pallas import tpu as pltpu

N = 10000
E = 320000
D = 128
DH = 16
EPS = 1e-5


def _node_in_body(x_ref, wi_ref, ws_ref, wd_ref, wl_ref, asrc_ref, adst_ref, v_ref):
    x = x_ref[...]
    y = jnp.dot(x, wi_ref[...].T, preferred_element_type=jnp.float32)
    mu = jnp.mean(y, axis=0, keepdims=True)
    var = jnp.mean(jnp.square(y - mu), axis=0, keepdims=True)
    h = jnp.maximum((y - mu) / jnp.sqrt(var + EPS), 0.0)
    asrc_ref[...] = jnp.dot(h, ws_ref[...].T, preferred_element_type=jnp.float32)
    adst_ref[...] = jnp.dot(h, wd_ref[...].T, preferred_element_type=jnp.float32)
    v_ref[...] = jnp.dot(h, wl_ref[...].T, preferred_element_type=jnp.float32)


@jax.jit
def _node_in(x, W_lin_in, W_src, W_dst, W_lin):
    return pl.pallas_call(
        _node_in_body,
        out_shape=[jax.ShapeDtypeStruct((N, D), jnp.float32)] * 3,
    )(x, W_lin_in, W_src, W_dst, W_lin)


def _node_out_body(osum_ref, ssum_ref, xskip_ref, wo_ref, out_ref):
    out = osum_ref[...] / (ssum_ref[...] + 1e-16)
    mu = jnp.mean(out, axis=0, keepdims=True)
    var = jnp.mean(jnp.square(out - mu), axis=0, keepdims=True)
    h2 = jnp.maximum((out - mu) / jnp.sqrt(var + EPS), 0.0)
    y = jnp.dot(h2, wo_ref[...].T, preferred_element_type=jnp.float32)
    mu2 = jnp.mean(y, axis=0, keepdims=True)
    var2 = jnp.mean(jnp.square(y - mu2), axis=0, keepdims=True)
    h3 = (y - mu2) / jnp.sqrt(var2 + EPS)
    out_ref[...] = jnp.maximum(h3 + xskip_ref[...], 0.0)


@jax.jit
def _node_out(out_sum, s_sum, x_skip, W_lin_out):
    return pl.pallas_call(
        _node_out_body,
        out_shape=jax.ShapeDtypeStruct((N, D), jnp.float32),
    )(out_sum, s_sum, x_skip, W_lin_out)


def kernel(x, pos, edge_index, W_lin_in, W_lin, W_src, W_dst, pos_W1, pos_b1,
           pos_W2, pos_b2, attn_W1, attn_b1, attn_W2, attn_b2, W_lin_out):
    src0 = edge_index[0]
    dst0 = edge_index[1]
    loops = jnp.arange(N, dtype=src0.dtype)
    src = jnp.concatenate([src0, loops])
    dst = jnp.concatenate([dst0, loops])
    vb = jnp.concatenate([src0 != dst0, jnp.ones((N,), dtype=bool)])
    w = vb.astype(jnp.float32)

    a_src, a_dst, v = _node_in(x, W_lin_in, W_src, W_dst, W_lin)

    sw = w.sum()

    def bn_w(t):
        mu = (t * w[:, None]).sum(axis=0) / sw
        var = (jnp.square(t - mu) * w[:, None]).sum(axis=0) / sw
        return (t - mu) / jnp.sqrt(var + EPS)

    pd = pos[dst] - pos[src]
    d1 = jnp.maximum(bn_w(pd @ pos_W1.T + pos_b1), 0.0)
    delta = d1 @ pos_W2.T + pos_b2
    alpha = a_dst[dst] - a_src[src] + delta
    a = jnp.maximum(bn_w(alpha), 0.0)
    a = jnp.maximum(bn_w(a @ attn_W1.T + attn_b1), 0.0)
    a = a @ attn_W2.T + attn_b2
    c = a[E + dst]
    e = jnp.exp(a - c) * w[:, None]
    s_sum = jax.ops.segment_sum(e, dst, num_segments=N)
    msg = e * (v[src] + delta)
    out_sum = jax.ops.segment_sum(msg, dst, num_segments=N)

    return _node_out(out_sum, s_sum, x, W_lin_out)


# speedup vs baseline: 1.9285x; 1.4125x over previous
import jax
import jax.numpy as jnp
from jax import lax
from jax.experimental import pallas as pl
from jax.experimental.pallas import tpu as pltpu
from jax.experimental.pallas import tpu_sc as plsc

N = 10000
E = 320000
ET = E + N
D = 128
DH = 16
EPS = 1e-5

NC, NS, L = 2, 16, 16
K = 128
K5 = 96
T_A = 20736
T_B = 10368
TPAD = NS * T_A
DHF = D // 2
B = 2048

_sc_mesh = plsc.VectorSubcoreMesh(core_axis_name="c", subcore_axis_name="s")
_sc_params = pltpu.CompilerParams(use_tc_tiling_on_sc=False)


def _node_in_body(x_ref, wi_ref, ws_ref, wd_ref, wl_ref, asrc_ref, adst_ref, v_ref):
    x = x_ref[...]
    y = jnp.dot(x, wi_ref[...].T, preferred_element_type=jnp.float32)
    mu = jnp.mean(y, axis=0, keepdims=True)
    var = jnp.mean(jnp.square(y - mu), axis=0, keepdims=True)
    h = jnp.maximum((y - mu) / jnp.sqrt(var + EPS), 0.0)
    asrc_ref[...] = jnp.dot(h, ws_ref[...].T, preferred_element_type=jnp.float32)
    adst_ref[...] = jnp.dot(h, wd_ref[...].T, preferred_element_type=jnp.float32)
    v_ref[...] = jnp.dot(h, wl_ref[...].T, preferred_element_type=jnp.float32)


def _node_in(x, W_lin_in, W_src, W_dst, W_lin):
    return pl.pallas_call(
        _node_in_body,
        out_shape=[jax.ShapeDtypeStruct((N, D), jnp.float32)] * 3,
    )(x, W_lin_in, W_src, W_dst, W_lin)


def _node_out_body(osum_ref, ssum_ref, xskip_ref, wo_ref, out_ref):
    out = osum_ref[...] / (ssum_ref[...] + 1e-16)
    mu = jnp.mean(out, axis=0, keepdims=True)
    var = jnp.mean(jnp.square(out - mu), axis=0, keepdims=True)
    h2 = jnp.maximum((out - mu) / jnp.sqrt(var + EPS), 0.0)
    y = jnp.dot(h2, wo_ref[...].T, preferred_element_type=jnp.float32)
    mu2 = jnp.mean(y, axis=0, keepdims=True)
    var2 = jnp.mean(jnp.square(y - mu2), axis=0, keepdims=True)
    h3 = (y - mu2) / jnp.sqrt(var2 + EPS)
    out_ref[...] = jnp.maximum(h3 + xskip_ref[...], 0.0)


def _node_out(out_sum, s_sum, x_skip, W_lin_out):
    return pl.pallas_call(
        _node_out_body,
        out_shape=jax.ShapeDtypeStruct((N, D), jnp.float32),
    )(out_sum, s_sum, x_skip, W_lin_out)


def _extract_w1(sv_vm):
    r0 = sv_vm[0, :]
    W1 = [[r0[3 * r + k] for k in range(3)] for r in range(3)]
    b1 = [r0[9 + r] for r in range(3)]
    return W1, b1


def _d1_scalars(ps, pdb, e, W1, b1, mu1, iv1):
    pdv = pdb[e, :] - ps[e, :]
    px, py, pz = pdv[0], pdv[1], pdv[2]
    out = []
    for r in range(3):
        t = W1[r][0] * px + W1[r][1] * py + W1[r][2] * pz + b1[r]
        if mu1 is None:
            out.append(t)
        else:
            out.append(jnp.maximum((t - mu1[r]) * iv1[r], 0.0))
    return out


def _p1_body(src_h, dst_h, w_h, pos_h, sv_h, out_h,
             sv_vm, src_v, dst_v, w_v, ps, pdb, part, sem):
    c = lax.axis_index("c")
    s = lax.axis_index("s")
    wid = c * NS + s
    pltpu.sync_copy(sv_h, sv_vm)
    W1, b1 = _extract_w1(sv_vm)
    base0 = wid * T_B

    def chunk(k, carry):
        b = base0 + k * K
        pltpu.sync_copy(src_h.at[pl.ds(b, K)], src_v)
        pltpu.sync_copy(dst_h.at[pl.ds(b, K)], dst_v)
        pltpu.sync_copy(w_h.at[pl.ds(b, K)], w_v)
        pltpu.async_copy(pos_h.at[src_v], ps, sem).wait()
        pltpu.async_copy(pos_h.at[dst_v], pdb, sem).wait()

        def grp(g, cr):
            accw, a0, a1, a2, q0, q1, q2 = cr
            wvec = w_v[pl.ds(g * L, L)]
            for el in range(L):
                e = g * L + el
                t0, t1, t2 = _d1_scalars(ps, pdb, e, W1, b1, None, None)
                we = wvec[el]
                accw = accw + we
                a0 = a0 + we * t0
                a1 = a1 + we * t1
                a2 = a2 + we * t2
                q0 = q0 + we * t0 * t0
                q1 = q1 + we * t1 * t1
                q2 = q2 + we * t2 * t2
            return accw, a0, a1, a2, q0, q1, q2

        return lax.fori_loop(0, K // L, grp, carry)

    z = jnp.zeros((), jnp.float32)
    acc = lax.fori_loop(0, T_B // K, chunk, (z,) * 7)
    for r in range(7):
        part[r, :] = jnp.full((L,), acc[r], jnp.float32)
    part[7, :] = jnp.zeros((L,), jnp.float32)
    pltpu.sync_copy(part, out_h.at[wid])


def _p1(src_p, dst_p, w_p, pos16, sv1):
    return pl.kernel(
        _p1_body,
        out_type=jax.ShapeDtypeStruct((NC * NS, 8, L), jnp.float32),
        mesh=_sc_mesh,
        compiler_params=_sc_params,
        scratch_types=[
            pltpu.VMEM((2, L), jnp.float32),
            pltpu.VMEM((K,), jnp.int32),
            pltpu.VMEM((K,), jnp.int32),
            pltpu.VMEM((K,), jnp.float32),
            pltpu.VMEM((K, L), jnp.float32),
            pltpu.VMEM((K, L), jnp.float32),
            pltpu.VMEM((8, L), jnp.float32),
            pltpu.SemaphoreType.DMA,
        ],
    )(src_p, dst_p, w_p, pos16, sv1)


def _p2_body(asrc_h, adst_h, src_h, dst_h, w_h, pos_h, sv_h, w2t_h, b2r_h,
             alpha_h, st_h,
             tbls, tbld, sv_vm, w2_vm, b2_vm,
             src_v, dst_v, w_v, ps, pdb, rs, rd, ob, part, sem):
    c = lax.axis_index("c")
    s = lax.axis_index("s")
    wid = c * NS + s

    @pl.when(s == 0)
    def _():
        pltpu.sync_copy(asrc_h.at[pl.ds(c * N, N)], tbls)
        pltpu.sync_copy(adst_h.at[pl.ds(c * N, N)], tbld)

    pltpu.sync_copy(sv_h, sv_vm)
    pltpu.sync_copy(w2t_h.at[pl.ds(c * 3, 3)], w2_vm)
    pltpu.sync_copy(b2r_h.at[c], b2_vm)
    plsc.subcore_barrier()

    W1, b1 = _extract_w1(sv_vm)
    r1 = sv_vm[1, :]
    mu1 = [r1[r] for r in range(3)]
    iv1 = [r1[3 + r] for r in range(3)]
    nj = DHF // L
    w2v = [[w2_vm[kk, pl.ds(L * j, L)] for j in range(nj)] for kk in range(3)]
    b2v = [b2_vm[pl.ds(L * j, L)] for j in range(nj)]
    base0 = s * T_A

    def chunk(k, carry):
        b = base0 + k * K
        pltpu.sync_copy(src_h.at[pl.ds(b, K)], src_v)
        pltpu.sync_copy(dst_h.at[pl.ds(b, K)], dst_v)
        pltpu.sync_copy(w_h.at[pl.ds(b, K)], w_v)
        pltpu.async_copy(pos_h.at[src_v], ps, sem).wait()
        pltpu.async_copy(pos_h.at[dst_v], pdb, sem).wait()
        pltpu.async_copy(tbls.at[src_v], rs, sem).wait()
        pltpu.async_copy(tbld.at[dst_v], rd, sem).wait()

        def grp(g, cr):
            accs, accq = list(cr[0]), list(cr[1])
            wvec = w_v[pl.ds(g * L, L)]
            for el in range(L):
                e = g * L + el
                s0, s1, s2 = _d1_scalars(ps, pdb, e, W1, b1, mu1, iv1)
                wsc = wvec[el]
                for j in range(nj):
                    sl = pl.ds(L * j, L)
                    dl = b2v[j] + s0 * w2v[0][j] + s1 * w2v[1][j] + s2 * w2v[2][j]
                    al = rd[e, sl] - rs[e, sl] + dl
                    ob[e, sl] = al
                    t = wsc * al
                    accs[j] = accs[j] + t
                    accq[j] = accq[j] + t * al
            return tuple(accs), tuple(accq)

        carry = lax.fori_loop(0, K // L, grp, carry)
        pltpu.sync_copy(ob, alpha_h.at[pl.ds(c * TPAD + b, K)])
        return carry

    z = jnp.zeros((L,), jnp.float32)
    nj = DHF // L
    accs, accq = lax.fori_loop(0, T_A // K, chunk, ((z,) * nj, (z,) * nj))
    for j in range(nj):
        part[0, pl.ds(L * j, L)] = accs[j]
        part[1, pl.ds(L * j, L)] = accq[j]
    pltpu.sync_copy(part, st_h.at[wid])


def _p2(asrc2f, adst2f, src_p, dst_p, w_p, pos16, sv1, w2t, b2r):
    return pl.kernel(
        _p2_body,
        out_type=[
            jax.ShapeDtypeStruct((2 * TPAD, DHF), jnp.float32),
            jax.ShapeDtypeStruct((NC * NS, 2, DHF), jnp.float32),
        ],
        mesh=_sc_mesh,
        compiler_params=_sc_params,
        scratch_types=[
            pltpu.VMEM_SHARED((N, DHF), jnp.float32),
            pltpu.VMEM_SHARED((N, DHF), jnp.float32),
            pltpu.VMEM((2, L), jnp.float32),
            pltpu.VMEM((3, DHF), jnp.float32),
            pltpu.VMEM((DHF,), jnp.float32),
            pltpu.VMEM((K,), jnp.int32),
            pltpu.VMEM((K,), jnp.int32),
            pltpu.VMEM((K,), jnp.float32),
            pltpu.VMEM((K, L), jnp.float32),
            pltpu.VMEM((K, L), jnp.float32),
            pltpu.VMEM((K, DHF), jnp.float32),
            pltpu.VMEM((K, DHF), jnp.float32),
            pltpu.VMEM((K, DHF), jnp.float32),
            pltpu.VMEM((2, DHF), jnp.float32),
            pltpu.SemaphoreType.DMA,
        ],
    )(asrc2f, adst2f, src_p, dst_p, w_p, pos16, sv1, w2t, b2r)


def _p3_body(alo_ref, ahi_ref, w_ref, mu_ref, iv_ref, W1_ref, b1_ref,
             t2_ref, st_ref):
    al = jnp.concatenate([alo_ref[0], ahi_ref[0]], axis=1)
    a = jnp.maximum((al - mu_ref[...]) * iv_ref[...], 0.0)
    t2 = jnp.dot(a, W1_ref[...].T, preferred_element_type=jnp.float32) + b1_ref[...]
    t2_ref[...] = t2
    wv = w_ref[...]
    s0 = jnp.sum(t2 * wv, axis=0, keepdims=True)
    s1 = jnp.sum(t2 * t2 * wv, axis=0, keepdims=True)
    upd = jnp.concatenate([s0, s1, jnp.zeros((6, DH), jnp.float32)], axis=0)

    @pl.when(pl.program_id(0) == 0)
    def _():
        st_ref[...] = jnp.zeros((8, DH), jnp.float32)

    st_ref[...] += upd


def _p3(alpha2, w2d, mu2, iv2, attn_W1, attn_b1):
    nsteps = TPAD // B
    return pl.pallas_call(
        _p3_body,
        grid=(nsteps,),
        in_specs=[
            pl.BlockSpec((1, B, DHF), lambda i: (0, i, 0)),
            pl.BlockSpec((1, B, DHF), lambda i: (1, i, 0)),
            pl.BlockSpec((B, 1), lambda i: (i, 0)),
            pl.BlockSpec((1, D), lambda i: (0, 0)),
            pl.BlockSpec((1, D), lambda i: (0, 0)),
            pl.BlockSpec((DH, D), lambda i: (0, 0)),
            pl.BlockSpec((1, DH), lambda i: (0, 0)),
        ],
        out_specs=[
            pl.BlockSpec((B, DH), lambda i: (i, 0)),
            pl.BlockSpec((8, DH), lambda i: (0, 0)),
        ],
        out_shape=[
            jax.ShapeDtypeStruct((TPAD, DH), jnp.float32),
            jax.ShapeDtypeStruct((8, DH), jnp.float32),
        ],
    )(alpha2, alpha2, w2d, mu2, iv2, attn_W1, attn_b1)


def _p4_body(t2_ref, mu_ref, iv_ref, W2_ref, b2_ref, a3_ref):
    a2 = jnp.maximum((t2_ref[...] - mu_ref[...]) * iv_ref[...], 0.0)
    y = jnp.dot(a2, W2_ref[0].T, preferred_element_type=jnp.float32) + b2_ref[0]
    a3_ref[0] = y


def _p4(t2, mu3, iv3, W2r, b2a):
    nsteps = TPAD // B
    return pl.pallas_call(
        _p4_body,
        grid=(2, nsteps),
        in_specs=[
            pl.BlockSpec((B, DH), lambda h, i: (i, 0)),
            pl.BlockSpec((1, DH), lambda h, i: (0, 0)),
            pl.BlockSpec((1, DH), lambda h, i: (0, 0)),
            pl.BlockSpec((1, DHF, DH), lambda h, i: (h, 0, 0)),
            pl.BlockSpec((1, 1, DHF), lambda h, i: (h, 0, 0)),
        ],
        out_specs=pl.BlockSpec((1, B, DHF), lambda h, i: (h, i, 0)),
        out_shape=jax.ShapeDtypeStruct((2, TPAD, DHF), jnp.float32),
    )(t2, mu3, iv3, W2r, b2a)


def _p5_body(a3_h, vf_h, src_h, dst_h, w_h, pos_h, sv_h, w2t_h, b2r_h, zer_h,
             outm_h, outs_h,
             accm, accs_sh, sv_vm, w2_vm, b2_vm,
             src_v, dst_v, w_v, vix, cix, ps, pdb, ar, crw, vr, eb, mb, sem):
    c = lax.axis_index("c")
    s = lax.axis_index("s")

    @pl.when(s == 0)
    def _():
        pltpu.sync_copy(zer_h, accm)
        pltpu.sync_copy(zer_h, accs_sh)

    pltpu.sync_copy(sv_h, sv_vm)
    pltpu.sync_copy(w2t_h.at[pl.ds(c * 3, 3)], w2_vm)
    pltpu.sync_copy(b2r_h.at[c], b2_vm)
    plsc.subcore_barrier()

    W1, b1 = _extract_w1(sv_vm)
    r1 = sv_vm[1, :]
    mu1 = [r1[r] for r in range(3)]
    iv1 = [r1[3 + r] for r in range(3)]
    nj = DHF // L
    w2v = [[w2_vm[kk, pl.ds(L * j, L)] for j in range(nj)] for kk in range(3)]
    b2v = [b2_vm[pl.ds(L * j, L)] for j in range(nj)]
    base0 = s * T_A
    cN = c * N
    cstab = c * TPAD + E

    def chunk(k, carry):
        b = base0 + k * K5
        pltpu.sync_copy(src_h.at[pl.ds(b, K5)], src_v)
        pltpu.sync_copy(dst_h.at[pl.ds(b, K5)], dst_v)
        pltpu.sync_copy(w_h.at[pl.ds(b, K5)], w_v)

        def bld(g, _):
            sl = pl.ds(g * L, L)
            vix[sl] = src_v[sl] + cN
            cix[sl] = dst_v[sl] + cstab
            return 0

        lax.fori_loop(0, K5 // L, bld, 0)
        pltpu.async_copy(pos_h.at[src_v], ps, sem).wait()
        pltpu.async_copy(pos_h.at[dst_v], pdb, sem).wait()
        pltpu.async_copy(vf_h.at[vix], vr, sem).wait()
        pltpu.async_copy(a3_h.at[cix], crw, sem).wait()
        pltpu.sync_copy(a3_h.at[pl.ds(c * TPAD + b, K5)], ar)

        def grp(g, _):
            wvec = w_v[pl.ds(g * L, L)]
            for el in range(L):
                e = g * L + el
                s0, s1, s2 = _d1_scalars(ps, pdb, e, W1, b1, mu1, iv1)
                wsc = wvec[el]
                for j in range(nj):
                    sl = pl.ds(L * j, L)
                    dl = b2v[j] + s0 * w2v[0][j] + s1 * w2v[1][j] + s2 * w2v[2][j]
                    ex = jnp.exp(jnp.minimum(ar[e, sl] - crw[e, sl], 60.0)) * wsc
                    eb[e, sl] = ex
                    mb[e, sl] = ex * (vr[e, sl] + dl)
            return 0

        lax.fori_loop(0, K5 // L, grp, 0)
        pltpu.sync_copy(eb, accs_sh.at[dst_v], add=True)
        pltpu.sync_copy(mb, accm.at[dst_v], add=True)
        return carry

    lax.fori_loop(0, T_A // K5, chunk, 0)
    plsc.subcore_barrier()

    @pl.when(s == 0)
    def _():
        pltpu.sync_copy(accm, outm_h.at[c])
        pltpu.sync_copy(accs_sh, outs_h.at[c])


def _p5(a3f, vf, src_p, dst_p, w_p, pos16, sv1, w2t, b2r, zer):
    return pl.kernel(
        _p5_body,
        out_type=[
            jax.ShapeDtypeStruct((NC, N, DHF), jnp.float32),
            jax.ShapeDtypeStruct((NC, N, DHF), jnp.float32),
        ],
        mesh=_sc_mesh,
        compiler_params=_sc_params,
        scratch_types=[
            pltpu.VMEM_SHARED((N, DHF), jnp.float32),
            pltpu.VMEM_SHARED((N, DHF), jnp.float32),
            pltpu.VMEM((2, L), jnp.float32),
            pltpu.VMEM((3, DHF), jnp.float32),
            pltpu.VMEM((DHF,), jnp.float32),
            pltpu.VMEM((K5,), jnp.int32),
            pltpu.VMEM((K5,), jnp.int32),
            pltpu.VMEM((K5,), jnp.float32),
            pltpu.VMEM((K5,), jnp.int32),
            pltpu.VMEM((K5,), jnp.int32),
            pltpu.VMEM((K5, L), jnp.float32),
            pltpu.VMEM((K5, L), jnp.float32),
            pltpu.VMEM((K5, DHF), jnp.float32),
            pltpu.VMEM((K5, DHF), jnp.float32),
            pltpu.VMEM((K5, DHF), jnp.float32),
            pltpu.VMEM((K5, DHF), jnp.float32),
            pltpu.VMEM((K5, DHF), jnp.float32),
            pltpu.SemaphoreType.DMA,
        ],
    )(a3f, vf, src_p, dst_p, w_p, pos16, sv1, w2t, b2r, zer)


def kernel(x, pos, edge_index, W_lin_in, W_lin, W_src, W_dst, pos_W1, pos_b1,
           pos_W2, pos_b2, attn_W1, attn_b1, attn_W2, attn_b2, W_lin_out):
    f32 = jnp.float32
    src0 = edge_index[0]
    dst0 = edge_index[1]
    loops = jnp.arange(N, dtype=src0.dtype)
    padi = jnp.zeros((TPAD - ET,), src0.dtype)
    src_p = jnp.concatenate([src0, loops, padi]).astype(jnp.int32)
    dst_p = jnp.concatenate([dst0, loops, padi]).astype(jnp.int32)
    w_p = jnp.concatenate([(src0 != dst0).astype(f32), jnp.ones((N,), f32),
                           jnp.zeros((TPAD - ET,), f32)])

    pos16 = jnp.pad(pos, ((0, 0), (0, L - 3)))
    wrow0 = jnp.concatenate([pos_W1.reshape(-1), pos_b1, jnp.zeros((4,), f32)])

    a_src, a_dst, v = _node_in(x, W_lin_in, W_src, W_dst, W_lin)
    asrc2f = jnp.concatenate([a_src[:, :DHF], a_src[:, DHF:]], axis=0)
    adst2f = jnp.concatenate([a_dst[:, :DHF], a_dst[:, DHF:]], axis=0)
    vf = jnp.concatenate([v[:, :DHF], v[:, DHF:]], axis=0)

    sv1a = jnp.stack([wrow0, jnp.zeros((16,), f32)])
    parts1 = _p1(src_p, dst_p, w_p, pos16, sv1a)
    sw = jnp.sum(parts1[:, 0, 0])
    s1 = jnp.sum(parts1[:, 1:4, 0], axis=0)
    q1 = jnp.sum(parts1[:, 4:7, 0], axis=0)
    mu1 = s1 / sw
    iv1 = 1.0 / jnp.sqrt(q1 / sw - mu1 * mu1 + EPS)
    sv1 = jnp.stack([wrow0, jnp.concatenate([mu1, iv1, jnp.zeros((10,), f32)])])

    w2t = jnp.concatenate([pos_W2[:DHF].T, pos_W2[DHF:].T], axis=0)
    b2r = jnp.stack([pos_b2[:DHF], pos_b2[DHF:]])

    alphaf, parts2 = _p2(asrc2f, adst2f, src_p, dst_p, w_p, pos16, sv1, w2t, b2r)
    st2 = parts2.reshape(2, NS, 2, DHF).sum(axis=1)
    mu2 = jnp.concatenate([st2[0, 0], st2[1, 0]]) / sw
    q2 = jnp.concatenate([st2[0, 1], st2[1, 1]]) / sw
    iv2 = 1.0 / jnp.sqrt(q2 - mu2 * mu2 + EPS)

    alpha2 = alphaf.reshape(2, TPAD, DHF)
    t2, st3 = _p3(alpha2, w_p.reshape(TPAD, 1), mu2.reshape(1, D),
                  iv2.reshape(1, D), attn_W1, attn_b1.reshape(1, DH))
    mu3 = st3[0] / sw
    iv3 = 1.0 / jnp.sqrt(st3[1] / sw - mu3 * mu3 + EPS)

    W2r = jnp.stack([attn_W2[:DHF], attn_W2[DHF:]])
    b2a = jnp.stack([attn_b2[:DHF], attn_b2[DHF:]]).reshape(2, 1, DHF)
    a3 = _p4(t2, mu3.reshape(1, DH), iv3.reshape(1, DH), W2r, b2a)
    a3f = a3.reshape(2 * TPAD, DHF)

    zer = jnp.zeros((N, DHF), f32)
    outm, outs = _p5(a3f, vf, src_p, dst_p, w_p, pos16, sv1, w2t, b2r, zer)
    out_sum = jnp.concatenate([outm[0], outm[1]], axis=1)
    s_sum = jnp.concatenate([outs[0], outs[1]], axis=1)

    return _node_out(out_sum, s_sum, x, W_lin_out)
